# baseline (speedup 1.0000x reference)
import functools

import jax
import jax.numpy as jnp
from jax import lax
from jax.experimental import pallas as pl
from jax.experimental.pallas import tpu as pltpu
from jax.experimental.pallas import tpu_sc as plsc

KNB = 32
BN = 1024
SC_NC = 2
SC_NS = 16
SC_NW = SC_NC * SC_NS


def _proj_kernel(x_ref, wf_ref, bf_ref, wq_ref, bq_ref, wk_ref, wv_ref,
                 feat_ref, q_ref, kf_ref, vf_ref):
    x = x_ref[0]
    f = jnp.dot(x, wf_ref[...], preferred_element_type=jnp.float32) + bf_ref[...]
    feat_ref[0] = f
    q_ref[0] = jnp.dot(f, wq_ref[...], preferred_element_type=jnp.float32) + bq_ref[...]
    kf_ref[0] = jnp.dot(f, wk_ref[...], preferred_element_type=jnp.float32)
    vf_ref[0] = jnp.dot(f, wv_ref[...], preferred_element_type=jnp.float32)


def _fold_kernel(wp2_ref, wk_ref, wv_ref, bp2_ref, bv_ref,
                 wk2t_ref, wv2_ref, cv_ref):
    wp2 = wp2_ref[...]
    wk2t_ref[...] = lax.dot_general(
        wk_ref[...], wp2, (((0,), (1,)), ((), ())),
        preferred_element_type=jnp.float32)
    wv2_ref[...] = jnp.dot(wp2, wv_ref[...], preferred_element_type=jnp.float32)
    cv_ref[...] = jnp.dot(bp2_ref[...], wv_ref[...],
                          preferred_element_type=jnp.float32) + bv_ref[...]


def _extract_kernel(q_ref, kf_ref, c_ref, ct_ref,
                    qk_ref, kd_ref, mi_ref, qi_ref, xi_ref, yi_ref, zi_ref,
                    dist_s, *, n, dh, knb):
    q = q_ref[0]
    cn = c_ref[0]
    ct = ct_ref[0]
    scale = 1.0 / jnp.sqrt(jnp.float32(dh))

    cxn, cyn, czn = cn[:, 0:1], cn[:, 1:2], cn[:, 2:3]
    cxm, cym, czm = ct[0:1, :], ct[1:2, :], ct[2:3, :]

    dx = cxn - cxm
    dy = cyn - cym
    sq = dx * dx + dy * dy
    spatial = jnp.where(sq > 0, jnp.sqrt(jnp.where(sq > 0, sq, 1.0)), 0.0)
    dist = spatial + 0.3 * jnp.abs(czn - czm)

    b = pl.program_id(0)
    i = pl.program_id(1)
    rid = lax.broadcasted_iota(jnp.int32, (BN, 1), 0) + i * BN
    cid = lax.broadcasted_iota(jnp.int32, (1, n), 1)
    inf = jnp.float32(jnp.inf)
    dist = jnp.where((rid == cid) | (czm > czn), inf, dist)
    dist_s[...] = dist

    qk_ref[0] = lax.dot_general(q, kf_ref[0], (((1,), (1,)), ((), ())),
                                preferred_element_type=jnp.float32) * scale

    l32 = lax.broadcasted_iota(jnp.int32, (1, knb), 1)

    def body(k, carry):
        kpack, mpack = carry
        d = dist_s[...]
        rowmin = jnp.min(d, axis=1, keepdims=True)
        eq = d == rowmin
        dist_s[...] = jnp.where(eq, inf, d)
        midx = jnp.min(jnp.where(eq, cid, jnp.int32(n)), axis=1,
                       keepdims=True)
        sel = l32 == k
        kpack = jnp.where(sel, rowmin, kpack)
        mpack = jnp.where(sel, midx, mpack)
        return kpack, mpack

    kp0 = jnp.zeros((BN, knb), jnp.float32)
    mp0 = jnp.zeros((BN, knb), jnp.int32)
    kpack, mpack = lax.fori_loop(0, knb, body, (kp0, mp0))

    kd_ref[0] = kpack
    mi_ref[0] = mpack
    rown = rid
    qi_ref[0] = (b * n + rown) * n + mpack
    xbase = b * 3 * n
    xi_ref[0] = xbase + mpack
    yi_ref[0] = xbase + n + mpack
    zi_ref[0] = xbase + 2 * n + mpack


def _sc_gather(qkflat, cflat, qi, xi, yi, zi,
               sq_o, gx_o, gy_o, gz_o,
               qi_v, xi_v, yi_v, zi_v, sq_v, gx_v, gy_v, gz_v, sem):
    wid = lax.axis_index("s") * SC_NC + lax.axis_index("c")
    pltpu.sync_copy(qi.at[wid], qi_v)
    pltpu.sync_copy(xi.at[wid], xi_v)
    pltpu.sync_copy(yi.at[wid], yi_v)
    pltpu.sync_copy(zi.at[wid], zi_v)

    def body(j, carry):
        cp1 = pltpu.async_copy(qkflat.at[qi_v.at[j]], sq_v.at[j], sem)
        cp2 = pltpu.async_copy(cflat.at[xi_v.at[j]], gx_v.at[j], sem)
        cp3 = pltpu.async_copy(cflat.at[yi_v.at[j]], gy_v.at[j], sem)
        cp4 = pltpu.async_copy(cflat.at[zi_v.at[j]], gz_v.at[j], sem)
        cp1.wait()
        cp2.wait()
        cp3.wait()
        cp4.wait()
        return carry

    lax.fori_loop(0, qi_v.shape[0], body, 0)
    pltpu.sync_copy(sq_v, sq_o.at[wid])
    pltpu.sync_copy(gx_v, gx_o.at[wid])
    pltpu.sync_copy(gy_v, gy_o.at[wid])
    pltpu.sync_copy(gz_v, gz_o.at[wid])


def _pe_kernel(q_ref, c_ref, sq_ref, kd_ref, gx_ref, gy_ref, gz_ref,
               wk2t_ref, wv2_ref, wp1_ref, bp1_ref,
               u_ref, pv_ref, zs_ref, hn_ref, *, dh, knb):
    q = q_ref[0]
    cn = c_ref[0]
    scale = 1.0 / jnp.sqrt(jnp.float32(dh))
    m_blk = jnp.dot(q, wk2t_ref[...],
                    preferred_element_type=jnp.float32) * scale
    cxn, cyn, czn = cn[:, 0:1], cn[:, 1:2], cn[:, 2:3]

    w1x = wp1_ref[0:1, :]
    w1y = wp1_ref[1:2, :]
    w1z = wp1_ref[2:3, :]
    w1d = wp1_ref[3:4, :]
    bp1 = bp1_ref[...]

    sqv = sq_ref[0]
    kdv = kd_ref[0]
    gxv = gx_ref[0]
    gyv = gy_ref[0]
    gzv = gz_ref[0]
    inf = jnp.float32(jnp.inf)

    nh = wp1_ref.shape[1]
    l32 = lax.broadcasted_iota(jnp.int32, (1, knb), 1)

    def body(k, carry):
        z, hacc, upack = carry
        sel = l32 == k
        kdk = jnp.sum(jnp.where(sel, kdv, 0.0), axis=1, keepdims=True)
        valid = jnp.sum(jnp.where(sel, jnp.where(kdv < inf, 1.0, 0.0), 0.0),
                        axis=1, keepdims=True) > 0
        kd = jnp.where(valid, kdk, 0.0)
        rx = jnp.sum(jnp.where(sel, gxv, 0.0), axis=1, keepdims=True) - cxn
        ry = jnp.sum(jnp.where(sel, gyv, 0.0), axis=1, keepdims=True) - cyn
        rz = jnp.sum(jnp.where(sel, gzv, 0.0), axis=1, keepdims=True) - czn
        sqk = jnp.sum(jnp.where(sel, sqv, 0.0), axis=1, keepdims=True)
        a = rx * w1x + ry * w1y + rz * w1z + kd * w1d + bp1
        h = jnp.maximum(a, 0.0)
        s2 = jnp.sum(h * m_blk, axis=1, keepdims=True)
        u = jnp.where(valid, jnp.exp(sqk + s2), 0.0)
        upack = jnp.where(sel, u, upack)
        return z + u, hacc + u * h, upack

    z, hacc, upack = lax.fori_loop(0, knb, body, (
        jnp.zeros((BN, 1), jnp.float32),
        jnp.zeros((BN, nh), jnp.float32),
        jnp.zeros((BN, knb), jnp.float32)))
    u_ref[0] = upack
    pv_ref[0] = jnp.dot(hacc, wv2_ref[...], preferred_element_type=jnp.float32)
    has_nb = kdv[:, 0:1] < inf
    zs_ref[0] = jnp.where(has_nb, z, 1.0)
    hn_ref[0] = jnp.where(has_nb, 1.0, 0.0)


def _agg_kernel(feat_ref, vf_ref, u_ref, mi_ref, pv_ref, zs_ref, hn_ref,
                cv_ref, g_ref, b_ref, o_ref, pacc_s, *, n, knb):
    bf = feat_ref[0]
    uv = u_ref[0]
    mv = mi_ref[0]
    l32 = lax.broadcasted_iota(jnp.int32, (1, knb), 1)
    cw = 512
    for c in range(n // cw):
        cidc = lax.broadcasted_iota(jnp.int32, (1, cw), 1) + c * cw

        def body(k, acc):
            sel = l32 == k
            mk = jnp.sum(jnp.where(sel, mv, 0), axis=1, keepdims=True)
            uk = jnp.sum(jnp.where(sel, uv, 0.0), axis=1, keepdims=True)
            return acc + jnp.where(cidc == mk, uk, 0.0)

        pacc_s[:, c * cw:(c + 1) * cw] = lax.fori_loop(
            0, knb, body, jnp.zeros((BN, cw), jnp.float32))
    v = jnp.dot(pacc_s[...], vf_ref[0], preferred_element_type=jnp.float32)
    agg = (v + pv_ref[0]) / zs_ref[0] + cv_ref[...]
    e = jnp.where(hn_ref[0] > 0, bf + agg, bf)
    mu = jnp.mean(e, axis=1, keepdims=True)
    var = jnp.mean((e - mu) ** 2, axis=1, keepdims=True)
    o_ref[0] = (e - mu) / jnp.sqrt(var + 1e-5) * g_ref[...] + b_ref[...]


def kernel(features, coords, W_feat, b_feat, W_pos1, b_pos1, W_pos2, b_pos2,
           W_q, b_q, W_k, b_k, W_v, b_v, gamma, beta):
    bsz, n, din = features.shape
    dout = W_feat.shape[1]
    dh = W_pos2.shape[0]
    nb = n // BN
    f32 = jnp.float32
    i32 = jnp.int32

    coords_t = jnp.swapaxes(coords, 1, 2)
    wp1 = jnp.zeros((8, dh), f32).at[:4].set(W_pos1)
    bp1 = b_pos1.reshape(1, dh)
    bp2 = b_pos2.reshape(1, dout)
    bv = b_v.reshape(1, dout)
    bfeat = b_feat.reshape(1, dout)
    bq = b_q.reshape(1, dout)
    g2 = gamma.reshape(1, dout)
    be2 = beta.reshape(1, dout)

    feat, q, kf, vf = pl.pallas_call(
        _proj_kernel,
        out_shape=[jax.ShapeDtypeStruct((bsz, n, dout), f32)] * 4,
        grid=(bsz, nb),
        in_specs=[
            pl.BlockSpec((1, BN, din), lambda b, i: (b, i, 0)),
            pl.BlockSpec((din, dout), lambda b, i: (0, 0)),
            pl.BlockSpec((1, dout), lambda b, i: (0, 0)),
            pl.BlockSpec((dout, dout), lambda b, i: (0, 0)),
            pl.BlockSpec((1, dout), lambda b, i: (0, 0)),
            pl.BlockSpec((dout, dout), lambda b, i: (0, 0)),
            pl.BlockSpec((dout, dout), lambda b, i: (0, 0)),
        ],
        out_specs=[pl.BlockSpec((1, BN, dout), lambda b, i: (b, i, 0))] * 4,
    )(features, W_feat, bfeat, W_q, bq, W_k, W_v)

    wk2t, wv2, cv = pl.pallas_call(
        _fold_kernel,
        out_shape=[
            jax.ShapeDtypeStruct((dout, dh), f32),
            jax.ShapeDtypeStruct((dh, dout), f32),
            jax.ShapeDtypeStruct((1, dout), f32),
        ],
    )(W_pos2, W_k, W_v, bp2, bv)

    kblk = pl.BlockSpec((1, BN, KNB), lambda b, i: (b, i, 0))
    qk, kd, mi, qi, xi, yi, zi = pl.pallas_call(
        functools.partial(_extract_kernel, n=n, dh=dout, knb=KNB),
        out_shape=[
            jax.ShapeDtypeStruct((bsz, n, n), f32),
            jax.ShapeDtypeStruct((bsz, n, KNB), f32),
            jax.ShapeDtypeStruct((bsz, n, KNB), i32),
            jax.ShapeDtypeStruct((bsz, n, KNB), i32),
            jax.ShapeDtypeStruct((bsz, n, KNB), i32),
            jax.ShapeDtypeStruct((bsz, n, KNB), i32),
            jax.ShapeDtypeStruct((bsz, n, KNB), i32),
        ],
        grid=(bsz, nb),
        in_specs=[
            pl.BlockSpec((1, BN, dout), lambda b, i: (b, i, 0)),
            pl.BlockSpec((1, n, dout), lambda b, i: (b, 0, 0)),
            pl.BlockSpec((1, BN, 3), lambda b, i: (b, i, 0)),
            pl.BlockSpec((1, 3, n), lambda b, i: (b, 0, 0)),
        ],
        out_specs=[pl.BlockSpec((1, BN, n), lambda b, i: (b, i, 0)),
                   kblk, kblk, kblk, kblk, kblk, kblk],
        scratch_shapes=[pltpu.VMEM((BN, n), f32)],
    )(q, kf, coords, coords_t)

    total = bsz * n * KNB
    ch_rows = total // (SC_NW * 128)
    idx_view = (SC_NW, ch_rows, 128)
    qkflat = qk.reshape(bsz * n * n)
    cflat = coords_t.reshape(bsz * 3 * n)

    mesh = plsc.VectorSubcoreMesh(core_axis_name="c", subcore_axis_name="s")
    sc_out = jax.ShapeDtypeStruct(idx_view, f32)
    gather = pl.kernel(
        _sc_gather,
        mesh=mesh,
        out_type=[sc_out] * 4,
        scratch_types=[pltpu.VMEM((ch_rows, 128), i32)] * 4
                      + [pltpu.VMEM((ch_rows, 128), f32)] * 4
                      + [pltpu.SemaphoreType.DMA],
    )
    sq_g, gx_g, gy_g, gz_g = gather(
        qkflat, cflat,
        qi.reshape(idx_view), xi.reshape(idx_view),
        yi.reshape(idx_view), zi.reshape(idx_view))
    sq_g = sq_g.reshape(bsz, n, KNB)
    gx_g = gx_g.reshape(bsz, n, KNB)
    gy_g = gy_g.reshape(bsz, n, KNB)
    gz_g = gz_g.reshape(bsz, n, KNB)

    u, pv, zs, hn = pl.pallas_call(
        functools.partial(_pe_kernel, dh=dout, knb=KNB),
        out_shape=[
            jax.ShapeDtypeStruct((bsz, n, KNB), f32),
            jax.ShapeDtypeStruct((bsz, n, dout), f32),
            jax.ShapeDtypeStruct((bsz, n, 1), f32),
            jax.ShapeDtypeStruct((bsz, n, 1), f32),
        ],
        grid=(bsz, nb),
        in_specs=[
            pl.BlockSpec((1, BN, dout), lambda b, i: (b, i, 0)),
            pl.BlockSpec((1, BN, 3), lambda b, i: (b, i, 0)),
            kblk, kblk, kblk, kblk, kblk,
            pl.BlockSpec((dout, dh), lambda b, i: (0, 0)),
            pl.BlockSpec((dh, dout), lambda b, i: (0, 0)),
            pl.BlockSpec((8, dh), lambda b, i: (0, 0)),
            pl.BlockSpec((1, dh), lambda b, i: (0, 0)),
        ],
        out_specs=[kblk,
                   pl.BlockSpec((1, BN, dout), lambda b, i: (b, i, 0)),
                   pl.BlockSpec((1, BN, 1), lambda b, i: (b, i, 0)),
                   pl.BlockSpec((1, BN, 1), lambda b, i: (b, i, 0))],
    )(q, coords, sq_g, kd, gx_g, gy_g, gz_g, wk2t, wv2, wp1, bp1)

    out = pl.pallas_call(
        functools.partial(_agg_kernel, n=n, knb=KNB),
        out_shape=jax.ShapeDtypeStruct((bsz, n, dout), f32),
        grid=(bsz, nb),
        in_specs=[
            pl.BlockSpec((1, BN, dout), lambda b, i: (b, i, 0)),
            pl.BlockSpec((1, n, dout), lambda b, i: (b, 0, 0)),
            kblk,
            kblk,
            pl.BlockSpec((1, BN, dout), lambda b, i: (b, i, 0)),
            pl.BlockSpec((1, BN, 1), lambda b, i: (b, i, 0)),
            pl.BlockSpec((1, BN, 1), lambda b, i: (b, i, 0)),
            pl.BlockSpec((1, dout), lambda b, i: (0, 0)),
            pl.BlockSpec((1, dout), lambda b, i: (0, 0)),
            pl.BlockSpec((1, dout), lambda b, i: (0, 0)),
        ],
        out_specs=pl.BlockSpec((1, BN, dout), lambda b, i: (b, i, 0)),
        scratch_shapes=[pltpu.VMEM((BN, n), f32)],
    )(feat, vf, u, mi, pv, zs, hn, cv, g2, be2)

    return out


# speedup vs baseline: 1.3346x; 1.3346x over previous
import functools

import jax
import jax.numpy as jnp
from jax import lax
from jax.experimental import pallas as pl
from jax.experimental.pallas import tpu as pltpu
from jax.experimental.pallas import tpu_sc as plsc

KNB = 32
BN = 1024
SC_NC = 2
SC_NS = 16
SC_NW = SC_NC * SC_NS


def _proj_kernel(x_ref, wf_ref, bf_ref, wq_ref, bq_ref, wk_ref, wv_ref,
                 feat_ref, q_ref, kf_ref, vf_ref):
    x = x_ref[0]
    f = jnp.dot(x, wf_ref[...], preferred_element_type=jnp.float32) + bf_ref[...]
    feat_ref[0] = f
    q_ref[0] = jnp.dot(f, wq_ref[...], preferred_element_type=jnp.float32) + bq_ref[...]
    kf_ref[0] = jnp.dot(f, wk_ref[...], preferred_element_type=jnp.float32)
    vf_ref[0] = jnp.dot(f, wv_ref[...], preferred_element_type=jnp.float32)


def _fold_kernel(wp2_ref, wk_ref, wv_ref, bp2_ref, bv_ref,
                 wk2t_ref, wv2_ref, cv_ref):
    wp2 = wp2_ref[...]
    wk2t_ref[...] = lax.dot_general(
        wk_ref[...], wp2, (((0,), (1,)), ((), ())),
        preferred_element_type=jnp.float32)
    wv2_ref[...] = jnp.dot(wp2, wv_ref[...], preferred_element_type=jnp.float32)
    cv_ref[...] = jnp.dot(bp2_ref[...], wv_ref[...],
                          preferred_element_type=jnp.float32) + bv_ref[...]


def _extract_kernel(q_ref, kf_ref, c_ref, ct_ref,
                    qk_ref, kd_ref, mi_ref, qi_ref, xi_ref, yi_ref, zi_ref,
                    dist_s, *, n, dh, knb):
    q = q_ref[0]
    cn = c_ref[0]
    ct = ct_ref[0]
    scale = 1.0 / jnp.sqrt(jnp.float32(dh))

    cxn, cyn, czn = cn[:, 0:1], cn[:, 1:2], cn[:, 2:3]
    cxm, cym, czm = ct[0:1, :], ct[1:2, :], ct[2:3, :]

    dx = cxn - cxm
    dy = cyn - cym
    sq = dx * dx + dy * dy
    spatial = jnp.where(sq > 0, jnp.sqrt(jnp.where(sq > 0, sq, 1.0)), 0.0)
    dist = spatial + 0.3 * jnp.abs(czn - czm)

    b = pl.program_id(0)
    i = pl.program_id(1)
    rid = lax.broadcasted_iota(jnp.int32, (BN, 1), 0) + i * BN
    cid = lax.broadcasted_iota(jnp.int32, (1, n), 1)
    inf = jnp.float32(jnp.inf)
    dist = jnp.where((rid == cid) | (czm > czn), inf, dist)
    dist_s[...] = dist

    qk_ref[0] = lax.dot_general(q, kf_ref[0], (((1,), (1,)), ((), ())),
                                preferred_element_type=jnp.float32) * scale

    l32 = lax.broadcasted_iota(jnp.int32, (1, knb), 1)

    def body(k, carry):
        kpack, mpack = carry
        d = dist_s[...]
        rowmin = jnp.min(d, axis=1, keepdims=True)
        eq = d == rowmin
        dist_s[...] = jnp.where(eq, inf, d)
        midx = jnp.min(jnp.where(eq, cid, jnp.int32(n)), axis=1,
                       keepdims=True)
        sel = l32 == k
        kpack = jnp.where(sel, rowmin, kpack)
        mpack = jnp.where(sel, midx, mpack)
        return kpack, mpack

    kp0 = jnp.zeros((BN, knb), jnp.float32)
    mp0 = jnp.zeros((BN, knb), jnp.int32)
    kpack, mpack = lax.fori_loop(0, knb, body, (kp0, mp0))

    kd_ref[0] = kpack
    mi_ref[0] = mpack
    rown = rid
    qi_ref[0] = (b * n + rown) * n + mpack
    xbase = b * 3 * n
    xi_ref[0] = xbase + mpack
    yi_ref[0] = xbase + n + mpack
    zi_ref[0] = xbase + 2 * n + mpack


def _sc_gather(qkflat, cflat, qi, xi, yi, zi,
               sq_o, gx_o, gy_o, gz_o,
               qi_v, xi_v, yi_v, zi_v, sq_v, gx_v, gy_v, gz_v, sem):
    wid = lax.axis_index("s") * SC_NC + lax.axis_index("c")
    pltpu.sync_copy(qi.at[wid], qi_v)
    pltpu.sync_copy(xi.at[wid], xi_v)
    pltpu.sync_copy(yi.at[wid], yi_v)
    pltpu.sync_copy(zi.at[wid], zi_v)

    def body(j, carry):
        cp1 = pltpu.async_copy(qkflat.at[qi_v.at[j]], sq_v.at[j], sem)
        cp2 = pltpu.async_copy(cflat.at[xi_v.at[j]], gx_v.at[j], sem)
        cp3 = pltpu.async_copy(cflat.at[yi_v.at[j]], gy_v.at[j], sem)
        cp4 = pltpu.async_copy(cflat.at[zi_v.at[j]], gz_v.at[j], sem)
        cp1.wait()
        cp2.wait()
        cp3.wait()
        cp4.wait()
        return carry

    lax.fori_loop(0, qi_v.shape[0], body, 0)
    pltpu.sync_copy(sq_v, sq_o.at[wid])
    pltpu.sync_copy(gx_v, gx_o.at[wid])
    pltpu.sync_copy(gy_v, gy_o.at[wid])
    pltpu.sync_copy(gz_v, gz_o.at[wid])


def _pe_kernel(q_ref, c_ref, sq_ref, kd_ref, gx_ref, gy_ref, gz_ref,
               wk2t_ref, wv2_ref, wp1_ref, bp1_ref,
               u_ref, pv_ref, zs_ref, hn_ref, *, dh, knb):
    q = q_ref[0]
    cn = c_ref[0]
    scale = 1.0 / jnp.sqrt(jnp.float32(dh))
    m_blk = jnp.dot(q, wk2t_ref[...],
                    preferred_element_type=jnp.float32) * scale
    cxn, cyn, czn = cn[:, 0:1], cn[:, 1:2], cn[:, 2:3]

    w1x = wp1_ref[0:1, :]
    w1y = wp1_ref[1:2, :]
    w1z = wp1_ref[2:3, :]
    w1d = wp1_ref[3:4, :]
    bp1 = bp1_ref[...]

    sqv = sq_ref[0]
    kdv = kd_ref[0]
    gxv = gx_ref[0]
    gyv = gy_ref[0]
    gzv = gz_ref[0]
    inf = jnp.float32(jnp.inf)

    nh = wp1_ref.shape[1]
    l32 = lax.broadcasted_iota(jnp.int32, (1, knb), 1)

    def body(k, carry):
        z, hacc, upack = carry
        sel = l32 == k
        kdk = jnp.sum(jnp.where(sel, kdv, 0.0), axis=1, keepdims=True)
        valid = jnp.sum(jnp.where(sel, jnp.where(kdv < inf, 1.0, 0.0), 0.0),
                        axis=1, keepdims=True) > 0
        kd = jnp.where(valid, kdk, 0.0)
        rx = jnp.sum(jnp.where(sel, gxv, 0.0), axis=1, keepdims=True) - cxn
        ry = jnp.sum(jnp.where(sel, gyv, 0.0), axis=1, keepdims=True) - cyn
        rz = jnp.sum(jnp.where(sel, gzv, 0.0), axis=1, keepdims=True) - czn
        sqk = jnp.sum(jnp.where(sel, sqv, 0.0), axis=1, keepdims=True)
        a = rx * w1x + ry * w1y + rz * w1z + kd * w1d + bp1
        h = jnp.maximum(a, 0.0)
        s2 = jnp.sum(h * m_blk, axis=1, keepdims=True)
        u = jnp.where(valid, jnp.exp(sqk + s2), 0.0)
        upack = jnp.where(sel, u, upack)
        return z + u, hacc + u * h, upack

    z, hacc, upack = lax.fori_loop(0, knb, body, (
        jnp.zeros((BN, 1), jnp.float32),
        jnp.zeros((BN, nh), jnp.float32),
        jnp.zeros((BN, knb), jnp.float32)))
    u_ref[0] = upack
    pv_ref[0] = jnp.dot(hacc, wv2_ref[...], preferred_element_type=jnp.float32)
    has_nb = kdv[:, 0:1] < inf
    zs_ref[0] = jnp.where(has_nb, z, 1.0)
    hn_ref[0] = jnp.where(has_nb, 1.0, 0.0)


def _agg_kernel(feat_ref, vf_ref, u_ref, mi_ref, pv_ref, zs_ref, hn_ref,
                cv_ref, g_ref, b_ref, o_ref, pacc_s, *, n, knb):
    bf = feat_ref[0]
    uv = u_ref[0]
    mv = mi_ref[0]
    cid = lax.broadcasted_iota(jnp.int32, (1, n), 1)
    l32 = lax.broadcasted_iota(jnp.int32, (1, knb), 1)
    pacc_s[...] = jnp.zeros((BN, n), jnp.float32)

    def body(k, carry):
        sel = l32 == k
        mk = jnp.sum(jnp.where(sel, mv, 0), axis=1, keepdims=True)
        uk = jnp.sum(jnp.where(sel, uv, 0.0), axis=1, keepdims=True)
        pacc_s[...] = pacc_s[...] + jnp.where(cid == mk, uk, 0.0)
        return carry

    lax.fori_loop(0, knb, body, 0)
    v = jnp.dot(pacc_s[...], vf_ref[0], preferred_element_type=jnp.float32)
    agg = (v + pv_ref[0]) / zs_ref[0] + cv_ref[...]
    e = jnp.where(hn_ref[0] > 0, bf + agg, bf)
    mu = jnp.mean(e, axis=1, keepdims=True)
    var = jnp.mean((e - mu) ** 2, axis=1, keepdims=True)
    o_ref[0] = (e - mu) / jnp.sqrt(var + 1e-5) * g_ref[...] + b_ref[...]


def kernel(features, coords, W_feat, b_feat, W_pos1, b_pos1, W_pos2, b_pos2,
           W_q, b_q, W_k, b_k, W_v, b_v, gamma, beta):
    bsz, n, din = features.shape
    dout = W_feat.shape[1]
    dh = W_pos2.shape[0]
    nb = n // BN
    f32 = jnp.float32
    i32 = jnp.int32

    coords_t = jnp.swapaxes(coords, 1, 2)
    wp1 = jnp.zeros((8, dh), f32).at[:4].set(W_pos1)
    bp1 = b_pos1.reshape(1, dh)
    bp2 = b_pos2.reshape(1, dout)
    bv = b_v.reshape(1, dout)
    bfeat = b_feat.reshape(1, dout)
    bq = b_q.reshape(1, dout)
    g2 = gamma.reshape(1, dout)
    be2 = beta.reshape(1, dout)

    feat, q, kf, vf = pl.pallas_call(
        _proj_kernel,
        out_shape=[jax.ShapeDtypeStruct((bsz, n, dout), f32)] * 4,
        grid=(bsz, nb),
        in_specs=[
            pl.BlockSpec((1, BN, din), lambda b, i: (b, i, 0)),
            pl.BlockSpec((din, dout), lambda b, i: (0, 0)),
            pl.BlockSpec((1, dout), lambda b, i: (0, 0)),
            pl.BlockSpec((dout, dout), lambda b, i: (0, 0)),
            pl.BlockSpec((1, dout), lambda b, i: (0, 0)),
            pl.BlockSpec((dout, dout), lambda b, i: (0, 0)),
            pl.BlockSpec((dout, dout), lambda b, i: (0, 0)),
        ],
        out_specs=[pl.BlockSpec((1, BN, dout), lambda b, i: (b, i, 0))] * 4,
    )(features, W_feat, bfeat, W_q, bq, W_k, W_v)

    wk2t, wv2, cv = pl.pallas_call(
        _fold_kernel,
        out_shape=[
            jax.ShapeDtypeStruct((dout, dh), f32),
            jax.ShapeDtypeStruct((dh, dout), f32),
            jax.ShapeDtypeStruct((1, dout), f32),
        ],
    )(W_pos2, W_k, W_v, bp2, bv)

    kblk = pl.BlockSpec((1, BN, KNB), lambda b, i: (b, i, 0))
    qk, kd, mi, qi, xi, yi, zi = pl.pallas_call(
        functools.partial(_extract_kernel, n=n, dh=dout, knb=KNB),
        out_shape=[
            jax.ShapeDtypeStruct((bsz, n, n), f32),
            jax.ShapeDtypeStruct((bsz, n, KNB), f32),
            jax.ShapeDtypeStruct((bsz, n, KNB), i32),
            jax.ShapeDtypeStruct((bsz, n, KNB), i32),
            jax.ShapeDtypeStruct((bsz, n, KNB), i32),
            jax.ShapeDtypeStruct((bsz, n, KNB), i32),
            jax.ShapeDtypeStruct((bsz, n, KNB), i32),
        ],
        grid=(bsz, nb),
        in_specs=[
            pl.BlockSpec((1, BN, dout), lambda b, i: (b, i, 0)),
            pl.BlockSpec((1, n, dout), lambda b, i: (b, 0, 0)),
            pl.BlockSpec((1, BN, 3), lambda b, i: (b, i, 0)),
            pl.BlockSpec((1, 3, n), lambda b, i: (b, 0, 0)),
        ],
        out_specs=[pl.BlockSpec((1, BN, n), lambda b, i: (b, i, 0)),
                   kblk, kblk, kblk, kblk, kblk, kblk],
        scratch_shapes=[pltpu.VMEM((BN, n), f32)],
    )(q, kf, coords, coords_t)

    total = bsz * n * KNB
    ch_rows = total // (SC_NW * 128)
    idx_view = (SC_NW, ch_rows, 128)
    qkflat = qk.reshape(bsz * n * n)
    cflat = coords_t.reshape(bsz * 3 * n)

    mesh = plsc.VectorSubcoreMesh(core_axis_name="c", subcore_axis_name="s")
    sc_out = jax.ShapeDtypeStruct(idx_view, f32)
    gather = pl.kernel(
        _sc_gather,
        mesh=mesh,
        out_type=[sc_out] * 4,
        scratch_types=[pltpu.VMEM((ch_rows, 128), i32)] * 4
                      + [pltpu.VMEM((ch_rows, 128), f32)] * 4
                      + [pltpu.SemaphoreType.DMA],
    )
    sq_g, gx_g, gy_g, gz_g = gather(
        qkflat, cflat,
        qi.reshape(idx_view), xi.reshape(idx_view),
        yi.reshape(idx_view), zi.reshape(idx_view))
    sq_g = sq_g.reshape(bsz, n, KNB)
    gx_g = gx_g.reshape(bsz, n, KNB)
    gy_g = gy_g.reshape(bsz, n, KNB)
    gz_g = gz_g.reshape(bsz, n, KNB)

    u, pv, zs, hn = pl.pallas_call(
        functools.partial(_pe_kernel, dh=dout, knb=KNB),
        out_shape=[
            jax.ShapeDtypeStruct((bsz, n, KNB), f32),
            jax.ShapeDtypeStruct((bsz, n, dout), f32),
            jax.ShapeDtypeStruct((bsz, n, 1), f32),
            jax.ShapeDtypeStruct((bsz, n, 1), f32),
        ],
        grid=(bsz, nb),
        in_specs=[
            pl.BlockSpec((1, BN, dout), lambda b, i: (b, i, 0)),
            pl.BlockSpec((1, BN, 3), lambda b, i: (b, i, 0)),
            kblk, kblk, kblk, kblk, kblk,
            pl.BlockSpec((dout, dh), lambda b, i: (0, 0)),
            pl.BlockSpec((dh, dout), lambda b, i: (0, 0)),
            pl.BlockSpec((8, dh), lambda b, i: (0, 0)),
            pl.BlockSpec((1, dh), lambda b, i: (0, 0)),
        ],
        out_specs=[kblk,
                   pl.BlockSpec((1, BN, dout), lambda b, i: (b, i, 0)),
                   pl.BlockSpec((1, BN, 1), lambda b, i: (b, i, 0)),
                   pl.BlockSpec((1, BN, 1), lambda b, i: (b, i, 0))],
    )(q, coords, sq_g, kd, gx_g, gy_g, gz_g, wk2t, wv2, wp1, bp1)

    out = pl.pallas_call(
        functools.partial(_agg_kernel, n=n, knb=KNB),
        out_shape=jax.ShapeDtypeStruct((bsz, n, dout), f32),
        grid=(bsz, nb),
        in_specs=[
            pl.BlockSpec((1, BN, dout), lambda b, i: (b, i, 0)),
            pl.BlockSpec((1, n, dout), lambda b, i: (b, 0, 0)),
            kblk,
            kblk,
            pl.BlockSpec((1, BN, dout), lambda b, i: (b, i, 0)),
            pl.BlockSpec((1, BN, 1), lambda b, i: (b, i, 0)),
            pl.BlockSpec((1, BN, 1), lambda b, i: (b, i, 0)),
            pl.BlockSpec((1, dout), lambda b, i: (0, 0)),
            pl.BlockSpec((1, dout), lambda b, i: (0, 0)),
            pl.BlockSpec((1, dout), lambda b, i: (0, 0)),
        ],
        out_specs=pl.BlockSpec((1, BN, dout), lambda b, i: (b, i, 0)),
        scratch_shapes=[pltpu.VMEM((BN, n), f32)],
    )(feat, vf, u, mi, pv, zs, hn, cv, g2, be2)

    return out
